# Initial kernel scaffold; baseline (speedup 1.0000x reference)
import functools

import jax
import jax.numpy as jnp
from jax import lax
from jax.experimental import pallas as pl
from jax.experimental.pallas import tpu as pltpu
from jax.experimental.pallas import tpu_sc as plsc

_N = 10000
_E = 320000
_D = 128
_NC = 2
_NS = 16
_NW = _NC * _NS
_EW = _E // _NW
_C = 80
_NCH = _EW // _C
_ROWS_PER_TILE = _N // _NS


def _make_agg_kernel():
  mesh = plsc.VectorSubcoreMesh(core_axis_name="c", subcore_axis_name="s")

  @functools.partial(
      pl.kernel,
      mesh=mesh,
      out_type=jax.ShapeDtypeStruct((_NC, _N, _D), jnp.float32),
      scratch_types=[
          pltpu.VMEM((_NCH, _C), jnp.int32),
          pltpu.VMEM((_NCH, _C), jnp.int32),
          pltpu.VMEM((_C, _D), jnp.float32),
          pltpu.VMEM_SHARED((_N, _D), jnp.float32),
          pltpu.SemaphoreType.DMA,
      ],
  )
  def agg_kernel(x_hbm, src_hbm, dst_hbm, zeros_hbm, out_hbm,
                 src_v, dst_v, rows_v, agg_sh, sem):
    c = lax.axis_index("c")
    s = lax.axis_index("s")
    wid = c * _NS + s
    pltpu.sync_copy(
        zeros_hbm.at[pl.ds(s * _ROWS_PER_TILE, _ROWS_PER_TILE)],
        agg_sh.at[pl.ds(s * _ROWS_PER_TILE, _ROWS_PER_TILE)])
    pltpu.sync_copy(src_hbm.at[pl.ds(wid * _NCH, _NCH)], src_v)
    pltpu.sync_copy(dst_hbm.at[pl.ds(wid * _NCH, _NCH)], dst_v)
    plsc.subcore_barrier()

    def body(j, carry):
      pltpu.async_copy(x_hbm.at[src_v.at[j]], rows_v, sem).wait()
      pltpu.sync_copy(rows_v, agg_sh.at[dst_v.at[j]], add=True)
      return carry

    lax.fori_loop(0, _NCH, body, 0, unroll=False)
    plsc.subcore_barrier()
    pltpu.sync_copy(
        agg_sh.at[pl.ds(s * _ROWS_PER_TILE, _ROWS_PER_TILE)],
        out_hbm.at[c, pl.ds(s * _ROWS_PER_TILE, _ROWS_PER_TILE)])

  return agg_kernel


_agg_kernel = _make_agg_kernel()


def _dense_body(last, x_ref, agg_ref, w1_ref, b1_ref, g_ref, bt_ref,
                w2_ref, b2_ref, og_ref, ob_ref, o_ref):
  h = x_ref[...] + agg_ref[0] + agg_ref[1]
  h = jnp.dot(h, w1_ref[...], preferred_element_type=jnp.float32) + b1_ref[...]
  m = jnp.mean(h, axis=0, keepdims=True)
  v = jnp.mean((h - m) * (h - m), axis=0, keepdims=True)
  h = (h - m) * (g_ref[...] * lax.rsqrt(v + 1e-5)) + bt_ref[...]
  h = jnp.maximum(h, 0.0)
  h = jnp.dot(h, w2_ref[...], preferred_element_type=jnp.float32) + b2_ref[...]
  if last:
    nrm = jnp.sqrt(jnp.sum(h * h, axis=1, keepdims=True))
    h = h / jnp.maximum(nrm, 1e-12)
  else:
    m2 = jnp.mean(h, axis=0, keepdims=True)
    v2 = jnp.mean((h - m2) * (h - m2), axis=0, keepdims=True)
    h = (h - m2) * (og_ref[...] * lax.rsqrt(v2 + 1e-5)) + ob_ref[...]
    h = jnp.maximum(h, 0.0)
  o_ref[...] = h


def _dense_layer(x, agg, w1, b1, g, bt, w2, b2, og, ob, last):
  return pl.pallas_call(
      functools.partial(_dense_body, last),
      out_shape=jax.ShapeDtypeStruct((_N, _D), jnp.float32),
  )(x, agg, w1, b1.reshape(1, _D), g.reshape(1, _D), bt.reshape(1, _D),
    w2, b2.reshape(1, _D), og.reshape(1, _D), ob.reshape(1, _D))


def kernel(x, edge_index,
           W1_0, b1_0, g_0, bt_0, W2_0, b2_0,
           W1_1, b1_1, g_1, bt_1, W2_1, b2_1,
           W1_2, b1_2, g_2, bt_2, W2_2, b2_2,
           og_0, ob_0, og_1, ob_1):
  src = edge_index[0].reshape(_NW * _NCH, _C)
  dst = edge_index[1].reshape(_NW * _NCH, _C)
  zeros = jnp.zeros((_N, _D), jnp.float32)
  params = [
      (W1_0, b1_0, g_0, bt_0, W2_0, b2_0, og_0, ob_0),
      (W1_1, b1_1, g_1, bt_1, W2_1, b2_1, og_1, ob_1),
      (W1_2, b1_2, g_2, bt_2, W2_2, b2_2, og_0, ob_0),
  ]
  for l in range(3):
    w1, b1, g, bt, w2, b2, og, ob = params[l]
    agg = _agg_kernel(x, src, dst, zeros)
    x = _dense_layer(x, agg, w1, b1, g, bt, w2, b2, og, ob, last=(l == 2))
  return x


# speedup vs baseline: 6.4686x; 6.4686x over previous
import functools

import jax
import jax.numpy as jnp
from jax import lax
from jax.experimental import pallas as pl
from jax.experimental.pallas import tpu as pltpu
from jax.experimental.pallas import tpu_sc as plsc

_N = 10000
_E = 320000
_D = 128
_NC = 2
_NS = 16
_NW = _NC * _NS
_EW = _E // _NW
_C = 80
_NCH = _EW // _C
_RPT = 624
_TAIL = _N - _RPT * _NS


@functools.cache
def _make_agg_kernel():
  mesh = plsc.VectorSubcoreMesh(core_axis_name="c", subcore_axis_name="s",
                                num_cores=_NC, num_subcores=_NS)

  @functools.partial(
      pl.kernel,
      mesh=mesh,
      out_type=jax.ShapeDtypeStruct((_NC, _N, _D), jnp.float32),
      scratch_types=[
          pltpu.VMEM((_NCH, _C), jnp.int32),
          pltpu.VMEM((_NCH, _C), jnp.int32),
          pltpu.VMEM((_C, _D), jnp.float32),
          pltpu.VMEM_SHARED((_N, _D), jnp.float32),
          pltpu.SemaphoreType.DMA,
      ],
  )
  def agg_kernel(x_hbm, src_hbm, dst_hbm, zeros_hbm, out_hbm,
                 src_v, dst_v, rows_v, agg_sh, sem):
    c = lax.axis_index("c")
    s = lax.axis_index("s")
    wid = c * _NS + s
    pltpu.sync_copy(
        zeros_hbm.at[pl.ds(s * _RPT, _RPT)],
        agg_sh.at[pl.ds(s * _RPT, _RPT)])

    @pl.when(s == _NS - 1)
    def _():
      pltpu.sync_copy(
          zeros_hbm.at[pl.ds(_RPT * _NS, _TAIL)],
          agg_sh.at[pl.ds(_RPT * _NS, _TAIL)])

    pltpu.sync_copy(src_hbm.at[wid], src_v)
    pltpu.sync_copy(dst_hbm.at[wid], dst_v)
    plsc.subcore_barrier()

    def body(j, carry):
      pltpu.async_copy(x_hbm.at[src_v.at[j]], rows_v, sem).wait()
      pltpu.sync_copy(rows_v, agg_sh.at[dst_v.at[j]], add=True)
      return carry

    lax.fori_loop(0, _NCH, body, 0, unroll=False)
    plsc.subcore_barrier()
    pltpu.sync_copy(
        agg_sh.at[pl.ds(s * _RPT, _RPT)],
        out_hbm.at[c, pl.ds(s * _RPT, _RPT)])

    @pl.when(s == _NS - 1)
    def _():
      pltpu.sync_copy(
          agg_sh.at[pl.ds(_RPT * _NS, _TAIL)],
          out_hbm.at[c, pl.ds(_RPT * _NS, _TAIL)])

  return agg_kernel


def _dense_body(last, x_ref, agg_ref, w1_ref, b1_ref, g_ref, bt_ref,
                w2_ref, b2_ref, og_ref, ob_ref, o_ref):
  h = x_ref[...] + agg_ref[0] + agg_ref[1]
  h = jnp.dot(h, w1_ref[...], preferred_element_type=jnp.float32) + b1_ref[...]
  m = jnp.mean(h, axis=0, keepdims=True)
  v = jnp.mean((h - m) * (h - m), axis=0, keepdims=True)
  h = (h - m) * (g_ref[...] * lax.rsqrt(v + 1e-5)) + bt_ref[...]
  h = jnp.maximum(h, 0.0)
  h = jnp.dot(h, w2_ref[...], preferred_element_type=jnp.float32) + b2_ref[...]
  if last:
    nrm = jnp.sqrt(jnp.sum(h * h, axis=1, keepdims=True))
    h = h / jnp.maximum(nrm, 1e-12)
  else:
    m2 = jnp.mean(h, axis=0, keepdims=True)
    v2 = jnp.mean((h - m2) * (h - m2), axis=0, keepdims=True)
    h = (h - m2) * (og_ref[...] * lax.rsqrt(v2 + 1e-5)) + ob_ref[...]
    h = jnp.maximum(h, 0.0)
  o_ref[...] = h


def _dense_layer(x, agg, w1, b1, g, bt, w2, b2, og, ob, last):
  return pl.pallas_call(
      functools.partial(_dense_body, last),
      out_shape=jax.ShapeDtypeStruct((_N, _D), jnp.float32),
  )(x, agg, w1, b1.reshape(1, _D), g.reshape(1, _D), bt.reshape(1, _D),
    w2, b2.reshape(1, _D), og.reshape(1, _D), ob.reshape(1, _D))


def kernel(x, edge_index,
           W1_0, b1_0, g_0, bt_0, W2_0, b2_0,
           W1_1, b1_1, g_1, bt_1, W2_1, b2_1,
           W1_2, b1_2, g_2, bt_2, W2_2, b2_2,
           og_0, ob_0, og_1, ob_1):
  src = edge_index[0].reshape(_NW, _NCH, _C)
  dst = edge_index[1].reshape(_NW, _NCH, _C)
  zeros = jnp.zeros((_N, _D), jnp.float32)
  params = [
      (W1_0, b1_0, g_0, bt_0, W2_0, b2_0, og_0, ob_0),
      (W1_1, b1_1, g_1, bt_1, W2_1, b2_1, og_1, ob_1),
      (W1_2, b1_2, g_2, bt_2, W2_2, b2_2, og_0, ob_0),
  ]
  for l in range(3):
    w1, b1, g, bt, w2, b2, og, ob = params[l]
    agg = _make_agg_kernel()(x, src, dst, zeros)
    x = _dense_layer(x, agg, w1, b1, g, bt, w2, b2, og, ob, last=(l == 2))
  return x


# speedup vs baseline: 10.3936x; 1.6068x over previous
import functools

import jax
import jax.numpy as jnp
from jax import lax
from jax.experimental import pallas as pl
from jax.experimental.pallas import tpu as pltpu
from jax.experimental.pallas import tpu_sc as plsc

_N = 10000
_E = 320000
_D = 128
_NC = 2
_NS = 16
_NW = _NC * _NS
_EW = _E // _NW
_C = 80
_NCH = _EW // _C
_RPT = 624
_TAIL = _N - _RPT * _NS


@functools.cache
def _make_agg_kernel():
  mesh = plsc.VectorSubcoreMesh(core_axis_name="c", subcore_axis_name="s",
                                num_cores=_NC, num_subcores=_NS)

  @functools.partial(
      pl.kernel,
      mesh=mesh,
      out_type=jax.ShapeDtypeStruct((_NC, _N, _D), jnp.float32),
      scratch_types=[
          pltpu.VMEM((_EW,), jnp.int32),
          pltpu.VMEM((_NCH, _C), jnp.int32),
          pltpu.VMEM((_C, _D), jnp.float32),
          pltpu.VMEM((_C, _D), jnp.float32),
          pltpu.VMEM_SHARED((_N, _D), jnp.float32),
          pltpu.SemaphoreType.DMA,
          pltpu.SemaphoreType.DMA,
      ],
  )
  def agg_kernel(x_hbm, src_hbm, dst_hbm, zeros_hbm, out_hbm,
                 src_v, dst_v, rows0, rows1, agg_sh, sem0, sem1):
    c = lax.axis_index("c")
    s = lax.axis_index("s")
    wid = c * _NS + s
    pltpu.sync_copy(
        zeros_hbm.at[pl.ds(s * _RPT, _RPT)],
        agg_sh.at[pl.ds(s * _RPT, _RPT)])

    @pl.when(s == _NS - 1)
    def _():
      pltpu.sync_copy(
          zeros_hbm.at[pl.ds(_RPT * _NS, _TAIL)],
          agg_sh.at[pl.ds(_RPT * _NS, _TAIL)])

    pltpu.sync_copy(src_hbm.at[pl.ds(wid * _EW, _EW)], src_v)
    pltpu.sync_copy(dst_hbm.at[wid], dst_v)
    plsc.subcore_barrier()

    def src_slice(j):
      return src_v.at[pl.ds(pl.multiple_of(j * _C, 8), _C)]

    pltpu.async_copy(x_hbm.at[src_slice(0)], rows0, sem0)
    pltpu.async_copy(x_hbm.at[src_slice(1)], rows1, sem1)

    def body(i, carry):
      for b, rows, sem in ((0, rows0, sem0), (1, rows1, sem1)):
        j = 2 * i + b
        pltpu.make_async_copy(x_hbm.at[src_slice(j)], rows, sem).wait()
        pltpu.sync_copy(rows, agg_sh.at[dst_v.at[j]], add=True)
        nxt = j + 2

        @pl.when(nxt < _NCH)
        def _(rows=rows, sem=sem, nxt=nxt):
          pltpu.async_copy(x_hbm.at[src_slice(nxt)], rows, sem)

      return carry

    lax.fori_loop(0, _NCH // 2, body, 0, unroll=False)
    if _NCH % 2:
      j = _NCH - 1
      pltpu.make_async_copy(x_hbm.at[src_slice(j)], rows0, sem0).wait()
      pltpu.sync_copy(rows0, agg_sh.at[dst_v.at[j]], add=True)
    plsc.subcore_barrier()
    pltpu.sync_copy(
        agg_sh.at[pl.ds(s * _RPT, _RPT)],
        out_hbm.at[c, pl.ds(s * _RPT, _RPT)])

    @pl.when(s == _NS - 1)
    def _():
      pltpu.sync_copy(
          agg_sh.at[pl.ds(_RPT * _NS, _TAIL)],
          out_hbm.at[c, pl.ds(_RPT * _NS, _TAIL)])

  return agg_kernel


def _dense_body(last, x_ref, agg_ref, w1_ref, b1_ref, g_ref, bt_ref,
                w2_ref, b2_ref, og_ref, ob_ref, o_ref):
  h = x_ref[...] + agg_ref[0] + agg_ref[1]
  h = jnp.dot(h, w1_ref[...], preferred_element_type=jnp.float32) + b1_ref[...]
  m = jnp.mean(h, axis=0, keepdims=True)
  v = jnp.mean((h - m) * (h - m), axis=0, keepdims=True)
  h = (h - m) * (g_ref[...] * lax.rsqrt(v + 1e-5)) + bt_ref[...]
  h = jnp.maximum(h, 0.0)
  h = jnp.dot(h, w2_ref[...], preferred_element_type=jnp.float32) + b2_ref[...]
  if last:
    nrm = jnp.sqrt(jnp.sum(h * h, axis=1, keepdims=True))
    h = h / jnp.maximum(nrm, 1e-12)
  else:
    m2 = jnp.mean(h, axis=0, keepdims=True)
    v2 = jnp.mean((h - m2) * (h - m2), axis=0, keepdims=True)
    h = (h - m2) * (og_ref[...] * lax.rsqrt(v2 + 1e-5)) + ob_ref[...]
    h = jnp.maximum(h, 0.0)
  o_ref[...] = h


def _dense_layer(x, agg, w1, b1, g, bt, w2, b2, og, ob, last):
  return pl.pallas_call(
      functools.partial(_dense_body, last),
      out_shape=jax.ShapeDtypeStruct((_N, _D), jnp.float32),
  )(x, agg, w1, b1.reshape(1, _D), g.reshape(1, _D), bt.reshape(1, _D),
    w2, b2.reshape(1, _D), og.reshape(1, _D), ob.reshape(1, _D))


def kernel(x, edge_index,
           W1_0, b1_0, g_0, bt_0, W2_0, b2_0,
           W1_1, b1_1, g_1, bt_1, W2_1, b2_1,
           W1_2, b1_2, g_2, bt_2, W2_2, b2_2,
           og_0, ob_0, og_1, ob_1):
  src = edge_index[0]
  dst = edge_index[1].reshape(_NW, _NCH, _C)
  zeros = jnp.zeros((_N, _D), jnp.float32)
  params = [
      (W1_0, b1_0, g_0, bt_0, W2_0, b2_0, og_0, ob_0),
      (W1_1, b1_1, g_1, bt_1, W2_1, b2_1, og_1, ob_1),
      (W1_2, b1_2, g_2, bt_2, W2_2, b2_2, og_0, ob_0),
  ]
  for l in range(3):
    w1, b1, g, bt, w2, b2, og, ob = params[l]
    agg = _make_agg_kernel()(x, src, dst, zeros)
    x = _dense_layer(x, agg, w1, b1, g, bt, w2, b2, og, ob, last=(l == 2))
  return x
